# baseline (speedup 1.0000x reference)
import functools

import jax
import jax.numpy as jnp
from jax.experimental import pallas as pl
from jax.experimental.pallas import tpu as pltpu
from jax.experimental.pallas import tpu_sc as plsc

B, S, D = 4, 8192, 1024
NC, NS = 2, 16
NW = NC * NS
RPW = S // NW

_mesh = plsc.VectorSubcoreMesh(core_axis_name="c", subcore_axis_name="s")


@functools.partial(
    pl.kernel,
    out_type=[jax.ShapeDtypeStruct((B, S + 1, D), jnp.float32)] * 2,
    mesh=_mesh,
    scratch_types=[
        pltpu.VMEM((B,), jnp.int32),
        pltpu.VMEM((B, D), jnp.float32),
        pltpu.SemaphoreType.DMA,
        pltpu.SemaphoreType.DMA,
    ],
    compiler_params=pltpu.CompilerParams(use_tc_tiling_on_sc=False),
)
def _sc_concat(x_hbm, idx_hbm, w_hbm, o1_hbm, o2_hbm, idx_v, rows_v, sem, gsem):
    c = jax.lax.axis_index("c")
    s = jax.lax.axis_index("s")
    wid = s * NC + c
    lo = wid * RPW
    handles = []
    for b in range(B):
        handles.append(pltpu.make_async_copy(
            x_hbm.at[b, pl.ds(lo, RPW)],
            o1_hbm.at[b, pl.ds(lo + 1, RPW)], sem))
        handles.append(pltpu.make_async_copy(
            x_hbm.at[b, pl.ds(lo, RPW)],
            o2_hbm.at[b, pl.ds(lo + 1, RPW)], sem))
    for h in handles:
        h.start()

    @pl.when(wid == 0)
    def _():
        pltpu.sync_copy(idx_hbm, idx_v)
        pltpu.async_copy(w_hbm.at[idx_v], rows_v, gsem).wait()
        for b in range(B):
            pltpu.sync_copy(rows_v.at[pl.ds(b, 1)], o1_hbm.at[b, pl.ds(0, 1)])
            pltpu.sync_copy(rows_v.at[pl.ds(b, 1)], o2_hbm.at[b, pl.ds(0, 1)])

    for h in handles:
        h.wait()


def kernel(x, tissue_vector, registry_weight):
    return tuple(_sc_concat(x, tissue_vector[:, 0], registry_weight))


# speedup vs baseline: 18.0076x; 18.0076x over previous
import functools

import jax
import jax.numpy as jnp
from jax import lax
from jax.experimental import pallas as pl
from jax.experimental.pallas import tpu as pltpu
from jax.experimental.pallas import tpu_sc as plsc

B, S, D = 4, 8192, 1024
NC, NS = 2, 16
NW = NC * NS
WIN = 32
WPB = 8

_mesh = plsc.VectorSubcoreMesh(core_axis_name="c", subcore_axis_name="s")


@functools.partial(
    pl.kernel,
    out_type=[jax.ShapeDtypeStruct((B, S + 1, D), jnp.float32)] * 2,
    mesh=_mesh,
    scratch_types=[
        pltpu.VMEM((WIN,), jnp.int32),
        pltpu.VMEM((WIN,), jnp.int32),
        pltpu.VMEM((WIN, D), jnp.float32),
        pltpu.VMEM((WIN, D), jnp.float32),
        pltpu.SemaphoreType.DMA,
        pltpu.SemaphoreType.DMA,
        pltpu.SemaphoreType.DMA,
        pltpu.SemaphoreType.DMA,
    ],
)
def _sc_bulk(x_hbm, o1_hbm, o2_hbm, idx0, idx1, buf0, buf1, g0, g1, s0, s1):
    c = lax.axis_index("c")
    s = lax.axis_index("s")
    wid = s * NC + c
    idxbufs = [idx0, idx1]
    bufs = [buf0, buf1]
    gsems = [g0, g1]
    ssems = [s0, s1]
    iota = lax.iota(jnp.int32, 16)
    T = B * WPB

    def build(t):
        b, win = divmod(t, WPB)
        base = 8 + 256 * wid + WIN * win
        lo = jnp.minimum(base, S - WIN)
        ib = idxbufs[t % 2]
        ib[pl.ds(0, 16)] = lo - 1 + iota
        ib[pl.ds(16, 16)] = lo + 15 + iota
        return b, lo

    gath = [None] * T
    los = [None] * T
    bs = [None] * T
    pend = [None, None]
    for t in range(T):
        cur = t % 2
        if t == 0:
            bs[0], los[0] = build(0)
            gath[0] = pltpu.make_async_copy(
                x_hbm.at[bs[0]].at[idxbufs[0]], bufs[0], gsems[0])
            gath[0].start()
        if t + 1 < T:
            nxt = (t + 1) % 2
            if pend[nxt] is not None:
                for h in pend[nxt]:
                    h.wait()
                pend[nxt] = None
            bs[t + 1], los[t + 1] = build(t + 1)
            gath[t + 1] = pltpu.make_async_copy(
                x_hbm.at[bs[t + 1]].at[idxbufs[nxt]], bufs[nxt], gsems[nxt])
            gath[t + 1].start()
        gath[t].wait()
        lo = pl.multiple_of(los[t], 8)
        h1 = pltpu.make_async_copy(
            bufs[cur], o1_hbm.at[bs[t]].at[pl.ds(lo, WIN)], ssems[cur])
        h2 = pltpu.make_async_copy(
            bufs[cur], o2_hbm.at[bs[t]].at[pl.ds(lo, WIN)], ssems[cur])
        h1.start()
        h2.start()
        pend[cur] = [h1, h2]
    for ps in pend:
        if ps is not None:
            for h in ps:
                h.wait()


def _head_body(idx_ref, x_ref, w_ref, o1b_ref, o2b_ref, o1_ref, o2_ref,
               hbuf, sem1, sem2):
    b = pl.program_id(0)
    g = pl.program_id(1)

    @pl.when(g == 0)
    def _():
        sub = idx_ref[b, 0] % 8
        wrows = lax.broadcasted_iota(jnp.int32, (8, D), 0)
        picked = jnp.where(wrows == sub, w_ref[...], 0.0)
        hbuf[0:1, :] = jnp.sum(picked, axis=0, keepdims=True)
        hbuf[1:8, :] = x_ref[0, 0:7, :]
        c1 = pltpu.make_async_copy(hbuf, o1_ref.at[b, pl.ds(0, 8)], sem1)
        c2 = pltpu.make_async_copy(hbuf, o2_ref.at[b, pl.ds(0, 8)], sem2)
        c1.start()
        c2.start()
        c1.wait()
        c2.wait()

    @pl.when(g == 1)
    def _():
        hbuf[0:1, :] = x_ref[0, 7:8, :]
        c1 = pltpu.make_async_copy(
            hbuf.at[pl.ds(0, 1)], o1_ref.at[b, pl.ds(S, 1)], sem1)
        c2 = pltpu.make_async_copy(
            hbuf.at[pl.ds(0, 1)], o2_ref.at[b, pl.ds(S, 1)], sem2)
        c1.start()
        c2.start()
        c1.wait()
        c2.wait()


def kernel(x, tissue_vector, registry_weight):
    o1b, o2b = _sc_bulk(x)
    out_sd = jax.ShapeDtypeStruct((B, S + 1, D), jnp.float32)
    grid_spec = pltpu.PrefetchScalarGridSpec(
        num_scalar_prefetch=1,
        grid=(B, 2),
        in_specs=[
            pl.BlockSpec((1, 8, D), lambda b, g, idx: (b, g * 1023, 0)),
            pl.BlockSpec((8, D), lambda b, g, idx: (idx[b, 0] // 8, 0)),
            pl.BlockSpec(memory_space=pl.ANY),
            pl.BlockSpec(memory_space=pl.ANY),
        ],
        out_specs=[
            pl.BlockSpec(memory_space=pl.ANY),
            pl.BlockSpec(memory_space=pl.ANY),
        ],
        scratch_shapes=[
            pltpu.VMEM((8, D), jnp.float32),
            pltpu.SemaphoreType.DMA,
            pltpu.SemaphoreType.DMA,
        ],
    )
    o1, o2 = pl.pallas_call(
        _head_body,
        grid_spec=grid_spec,
        out_shape=[out_sd, out_sd],
        input_output_aliases={3: 0, 4: 1},
    )(tissue_vector, x, registry_weight, o1b, o2b)
    return (o1, o2)


# speedup vs baseline: 18.0532x; 1.0025x over previous
import functools

import jax
import jax.numpy as jnp
from jax import lax
from jax.experimental import pallas as pl
from jax.experimental.pallas import tpu as pltpu
from jax.experimental.pallas import tpu_sc as plsc

B, S, D = 4, 8192, 1024
NC, NS = 2, 16
NW = NC * NS
WIN = 32
WPB = 8

_mesh = plsc.VectorSubcoreMesh(core_axis_name="c", subcore_axis_name="s")


def _emb_body(idx_ref, w_ref, out_ref):
    b = pl.program_id(0)
    sub = idx_ref[b, 0] % 8
    wrows = lax.broadcasted_iota(jnp.int32, (8, D), 0)
    picked = jnp.where(wrows == sub, w_ref[...], 0.0)
    out_ref[0:1, :] = jnp.sum(picked, axis=0, keepdims=True)


@functools.partial(
    pl.kernel,
    out_type=[jax.ShapeDtypeStruct((B, S + 1, D), jnp.float32)] * 2,
    mesh=_mesh,
    scratch_types=[
        pltpu.VMEM((WIN,), jnp.int32),
        pltpu.VMEM((WIN,), jnp.int32),
        pltpu.VMEM((16,), jnp.int32),
        pltpu.VMEM((WIN, D), jnp.float32),
        pltpu.VMEM((WIN, D), jnp.float32),
        pltpu.VMEM((16, D), jnp.float32),
        pltpu.SemaphoreType.DMA,
        pltpu.SemaphoreType.DMA,
        pltpu.SemaphoreType.DMA,
        pltpu.SemaphoreType.DMA,
        pltpu.SemaphoreType.DMA,
    ],
)
def _sc_all(x_hbm, emb_hbm, o1_hbm, o2_hbm,
            idx0, idx1, ih, buf0, buf1, ghead, g0, g1, s0, s1, hsem):
    c = lax.axis_index("c")
    s = lax.axis_index("s")
    wid = s * NC + c
    idxbufs = [idx0, idx1]
    bufs = [buf0, buf1]
    gsems = [g0, g1]
    ssems = [s0, s1]
    iota = lax.iota(jnp.int32, 16)
    T = B * WPB

    for b in range(B):
        @pl.when(wid == b)
        def _(b=b):
            ih[...] = jnp.minimum(jnp.maximum(iota - 1, 0), 6)
            g = pltpu.make_async_copy(x_hbm.at[b].at[ih], ghead, hsem)
            g.start()
            g.wait()
            e = pltpu.make_async_copy(
                emb_hbm.at[pl.ds(8 * b, 1)], ghead.at[pl.ds(0, 1)], hsem)
            e.start()
            e.wait()
            for o in (o1_hbm, o2_hbm):
                sc = pltpu.make_async_copy(
                    ghead.at[pl.ds(0, 8)], o.at[b].at[pl.ds(0, 8)], hsem)
                sc.start()
                sc.wait()

        @pl.when(wid == B + b)
        def _(b=b):
            ih[...] = iota * 0 + (S - 1)
            g = pltpu.make_async_copy(x_hbm.at[b].at[ih], ghead, hsem)
            g.start()
            g.wait()
            for o in (o1_hbm, o2_hbm):
                sc = pltpu.make_async_copy(
                    ghead.at[pl.ds(0, 1)], o.at[b].at[pl.ds(S, 1)], hsem)
                sc.start()
                sc.wait()

    def build(t):
        b, win = divmod(t, WPB)
        base = 8 + 256 * wid + WIN * win
        lo = jnp.minimum(base, S - WIN)
        ib = idxbufs[t % 2]
        ib[pl.ds(0, 16)] = lo - 1 + iota
        ib[pl.ds(16, 16)] = lo + 15 + iota
        return b, lo

    gath = [None] * T
    los = [None] * T
    bs = [None] * T
    pend = [None, None]
    for t in range(T):
        cur = t % 2
        if t == 0:
            bs[0], los[0] = build(0)
            gath[0] = pltpu.make_async_copy(
                x_hbm.at[bs[0]].at[idxbufs[0]], bufs[0], gsems[0])
            gath[0].start()
        if t + 1 < T:
            nxt = (t + 1) % 2
            if pend[nxt] is not None:
                for h in pend[nxt]:
                    h.wait()
                pend[nxt] = None
            bs[t + 1], los[t + 1] = build(t + 1)
            gath[t + 1] = pltpu.make_async_copy(
                x_hbm.at[bs[t + 1]].at[idxbufs[nxt]], bufs[nxt], gsems[nxt])
            gath[t + 1].start()
        gath[t].wait()
        lo = pl.multiple_of(los[t], 8)
        h1 = pltpu.make_async_copy(
            bufs[cur], o1_hbm.at[bs[t]].at[pl.ds(lo, WIN)], ssems[cur])
        h2 = pltpu.make_async_copy(
            bufs[cur], o2_hbm.at[bs[t]].at[pl.ds(lo, WIN)], ssems[cur])
        h1.start()
        h2.start()
        pend[cur] = [h1, h2]
    for ps in pend:
        if ps is not None:
            for h in ps:
                h.wait()


def kernel(x, tissue_vector, registry_weight):
    emb8 = pl.pallas_call(
        _emb_body,
        grid_spec=pltpu.PrefetchScalarGridSpec(
            num_scalar_prefetch=1,
            grid=(B,),
            in_specs=[
                pl.BlockSpec((8, D), lambda b, idx: (idx[b, 0] // 8, 0)),
            ],
            out_specs=pl.BlockSpec((8, D), lambda b, idx: (b, 0)),
        ),
        out_shape=jax.ShapeDtypeStruct((8 * B, D), jnp.float32),
    )(tissue_vector, registry_weight)
    o1, o2 = _sc_all(x, emb8)
    return (o1, o2)


# speedup vs baseline: 60.7460x; 3.3648x over previous
import jax
import jax.numpy as jnp
from jax import lax
from jax.experimental import pallas as pl
from jax.experimental.pallas import tpu as pltpu

B, S, D = 4, 8192, 1024
CB = 512
NJX = S // CB
NJ = NJX + 1


def _body(idx_ref, x_ref, w_ref, o1_ref, o2_ref, carry_ref):
    j = pl.program_id(0)
    b = pl.program_id(1)

    for kb in range(B):
        @pl.when(b == kb)
        def _(kb=kb):
            @pl.when(j == 0)
            def _():
                sub = idx_ref[kb, 0] % 8
                wrows = lax.broadcasted_iota(jnp.int32, (8, D), 0)
                picked = jnp.where(wrows == sub, w_ref[...], 0.0)
                carry_ref[kb:kb + 1, :] = jnp.sum(picked, axis=0,
                                                  keepdims=True)

            blk = x_ref[0]
            shifted = pltpu.roll(blk, 1, 0)
            o1_ref[:, kb, :] = shifted
            o2_ref[:, kb, :] = shifted
            first = carry_ref[kb:kb + 1, :]
            o1_ref[0:1, kb, :] = first
            o2_ref[0:1, kb, :] = first
            carry_ref[kb:kb + 1, :] = blk[CB - 1:CB, :]


def kernel(x, tissue_vector, registry_weight):
    out_sd = jax.ShapeDtypeStruct((S + 1, B, D), jnp.float32)
    grid_spec = pltpu.PrefetchScalarGridSpec(
        num_scalar_prefetch=1,
        grid=(NJ, B),
        in_specs=[
            pl.BlockSpec((1, CB, D),
                         lambda j, b, idx: (b, jnp.minimum(j, NJX - 1), 0)),
            pl.BlockSpec((8, D), lambda j, b, idx: (idx[b, 0] // 8, 0)),
        ],
        out_specs=[
            pl.BlockSpec((CB, B, D), lambda j, b, idx: (j, 0, 0)),
            pl.BlockSpec((CB, B, D), lambda j, b, idx: (j, 0, 0)),
        ],
        scratch_shapes=[pltpu.VMEM((B, D), jnp.float32)],
    )
    o1t, o2t = pl.pallas_call(
        _body,
        grid_spec=grid_spec,
        out_shape=[out_sd, out_sd],
    )(tissue_vector, x, registry_weight)
    return (jnp.transpose(o1t, (1, 0, 2)), jnp.transpose(o2t, (1, 0, 2)))
